# Initial kernel scaffold; baseline (speedup 1.0000x reference)
import jax
import jax.numpy as jnp
from jax.experimental import pallas as pl


def kernel(rpn_bbox_deltas, rpn_probs, gt_labels, anchors):
    raise NotImplementedError("write your pallas kernel here")


# speedup vs baseline: 41.0327x; 41.0327x over previous
import jax
import jax.numpy as jnp
from jax.experimental import pallas as pl

_PRE_NMS = 6000
_POST_NMS = 1500
_IOU_THR = 0.7
_VARIANCES = (0.1, 0.1, 0.2, 0.2)
_ROWS = 8
_COLS = 768
_PAD = _ROWS * _COLS


def _nms_kernel(anch_ref, delt_ref, box_ref, keep_ref):
    y1a = anch_ref[0, 0]
    x1a = anch_ref[0, 1]
    y2a = anch_ref[0, 2]
    x2a = anch_ref[0, 3]
    d0 = delt_ref[0, 0]
    d1 = delt_ref[0, 1]
    d2 = delt_ref[0, 2]
    d3 = delt_ref[0, 3]

    aw = x2a - x1a
    ah = y2a - y1a
    acx = x1a + 0.5 * aw
    acy = y1a + 0.5 * ah
    bw = jnp.exp(d3) * aw
    bh = jnp.exp(d2) * ah
    bcx = d1 * aw + acx
    bcy = d0 * ah + acy
    y1 = bcy - 0.5 * bh
    x1 = bcx - 0.5 * bw
    y2 = y1 + bh
    x2 = x1 + bw

    box_ref[0, 0] = y1
    box_ref[0, 1] = x1
    box_ref[0, 2] = y2
    box_ref[0, 3] = x2

    area = jnp.maximum(y2 - y1, 0.0) * jnp.maximum(x2 - x1, 0.0)

    row = jax.lax.broadcasted_iota(jnp.int32, (_ROWS, _COLS), 0)
    col = jax.lax.broadcasted_iota(jnp.int32, (_ROWS, _COLS), 1)
    lin = row * _COLS + col

    neg = jnp.float32(-3.0e38)

    def body(i, keep):
        m = lin == i
        alive = jnp.max(jnp.where(m, keep, 0.0)) > 0.5
        y1i = jnp.max(jnp.where(m, y1, neg))
        x1i = jnp.max(jnp.where(m, x1, neg))
        y2i = jnp.max(jnp.where(m, y2, neg))
        x2i = jnp.max(jnp.where(m, x2, neg))
        ai = jnp.max(jnp.where(m, area, neg))

        inter = (jnp.maximum(jnp.minimum(y2i, y2) - jnp.maximum(y1i, y1), 0.0)
                 * jnp.maximum(jnp.minimum(x2i, x2) - jnp.maximum(x1i, x1), 0.0))
        union = ai + area - inter
        iou = inter / jnp.maximum(union, 1e-8)
        suppress = (iou > _IOU_THR) & (lin > i) & alive
        return jnp.where(suppress, 0.0, keep)

    keep = jax.lax.fori_loop(
        0, _PRE_NMS, body, jnp.ones((_ROWS, _COLS), jnp.float32))
    keep_ref[0] = keep


def kernel(rpn_bbox_deltas, rpn_probs, gt_labels, anchors):
    del gt_labels
    B = rpn_bbox_deltas.shape[0]
    A = anchors.shape[0]
    variances = jnp.asarray(_VARIANCES, dtype=jnp.float32)

    deltas = rpn_bbox_deltas.reshape(B, A, 4) * variances
    probs = rpn_probs.reshape(B, A)

    pre_scores, pre_idx = jax.lax.top_k(probs, _PRE_NMS)
    anch_sel = anchors[pre_idx].transpose(0, 2, 1)
    delt_sel = jnp.take_along_axis(
        deltas, pre_idx[:, :, None], axis=1).transpose(0, 2, 1)

    pad = ((0, 0), (0, 0), (0, _PAD - _PRE_NMS))
    anch_t = jnp.pad(anch_sel, pad).reshape(B, 4, _ROWS, _COLS)
    delt_t = jnp.pad(delt_sel, pad).reshape(B, 4, _ROWS, _COLS)

    box_t, keep_t = pl.pallas_call(
        _nms_kernel,
        grid=(B,),
        in_specs=[
            pl.BlockSpec((1, 4, _ROWS, _COLS), lambda b: (b, 0, 0, 0)),
            pl.BlockSpec((1, 4, _ROWS, _COLS), lambda b: (b, 0, 0, 0)),
        ],
        out_specs=[
            pl.BlockSpec((1, 4, _ROWS, _COLS), lambda b: (b, 0, 0, 0)),
            pl.BlockSpec((1, _ROWS, _COLS), lambda b: (b, 0, 0)),
        ],
        out_shape=[
            jax.ShapeDtypeStruct((B, 4, _ROWS, _COLS), jnp.float32),
            jax.ShapeDtypeStruct((B, _ROWS, _COLS), jnp.float32),
        ],
    )(anch_t, delt_t)

    boxes = box_t.reshape(B, 4, _PAD)[:, :, :_PRE_NMS].transpose(0, 2, 1)
    keep = keep_t.reshape(B, _PAD)[:, :_PRE_NMS] > 0.5

    masked = jnp.where(keep, pre_scores, -jnp.inf)
    sel_scores, sel_idx = jax.lax.top_k(masked, _POST_NMS)
    valid = jnp.isfinite(sel_scores)
    sel_boxes = jnp.clip(
        jnp.take_along_axis(boxes, sel_idx[:, :, None], axis=1), 0.0, 1.0)
    roi_bboxes = jnp.where(valid[:, :, None], sel_boxes, 0.0)
    roi_scores = jnp.where(valid, sel_scores, 0.0)
    return roi_bboxes, roi_scores
